# baseline (speedup 1.0000x reference)
import functools

import jax
import jax.numpy as jnp
from jax import lax
from jax.experimental import pallas as pl
from jax.experimental.pallas import tpu as pltpu
from jax.experimental.pallas import tpu_sc as plsc

N = 10000
E = 160000
D = 128
L = 16
NC = 2
NS = 16
C = 128
E_PAD = 163840
PH = 2
NSUB = E_PAD // (NS * C)
NSUB_PH = NSUB // PH
NCR = 5000
DUMMY = NCR
NCP = 5120
RPT = NCP // NS
WB = 80


def _seg_sum_body(table_hbm, src_hbm, dst_hbm, out_hbm, deg_hbm,
                  src_v, dst_v, rows_v, rows_w, ones_v, dstg_v,
                  acc_sh, deg_sh, gsem, dsem):
    cid = lax.axis_index("c")
    sid = lax.axis_index("s")

    def _zrow(i, carry):
        for k in range(D // L):
            rows_v[i, pl.ds(k * L, L)] = jnp.zeros((L,), jnp.float32)
        return carry
    lax.fori_loop(0, C, _zrow, 0)

    def _zsmall(i, carry):
        dstg_v[pl.ds(i * L, L)] = jnp.zeros((L,), jnp.float32)
        ones_v[pl.ds(i * L, L)] = jnp.ones((L,), jnp.float32)
        return carry
    lax.fori_loop(0, RPT // L, _zsmall, 0)

    def _zcopy(t, carry):
        r = sid * RPT + t * WB
        pltpu.sync_copy(rows_v.at[pl.ds(0, WB)], acc_sh.at[pl.ds(r, WB)])
        return carry
    lax.fori_loop(0, RPT // WB, _zcopy, 0)
    pltpu.sync_copy(dstg_v, deg_sh.at[pl.ds(sid * RPT, RPT)])

    plsc.subcore_barrier()

    base = (cid * NCR).astype(jnp.int32)
    for phase in range(PH):
        row0 = sid * NSUB + phase * NSUB_PH
        pltpu.sync_copy(src_hbm.at[pl.ds(row0, NSUB_PH)], src_v)
        pltpu.sync_copy(dst_hbm.at[pl.ds(row0, NSUB_PH)], dst_v)

        lane = lax.iota(jnp.int32, L)

        def _adj(j, carry):
            for k in range(C // L):
                d = dst_v[j, pl.ds(k * L, L)] - base
                ok = (d >= 0) & (d < NCR)
                dummy = DUMMY + ((lane * 7 + j * 13 + (sid + k) * L)
                                 % (NCP - NCR))
                dst_v[j, pl.ds(k * L, L)] = jnp.where(ok, d, dummy)
            return carry
        lax.fori_loop(0, NSUB_PH, _adj, 0)

        pltpu.async_copy(table_hbm.at[src_v.at[0]], rows_v, gsem)

        def _pair(p, carry):
            j0 = p * 2
            j1 = j0 + 1
            pltpu.async_copy(table_hbm.at[src_v.at[j1]], rows_w, gsem)
            pltpu.make_async_copy(table_hbm.at[src_v.at[j0]],
                                  rows_v, gsem).wait()
            pltpu.sync_copy(rows_v, acc_sh.at[dst_v.at[j0]], add=True)
            pltpu.async_copy(ones_v.at[pl.ds(0, C)],
                             deg_sh.at[dst_v.at[j0]], dsem, add=True)
            j2 = jnp.minimum(j0 + 2, NSUB_PH - 1)
            pltpu.async_copy(table_hbm.at[src_v.at[j2]], rows_v, gsem)
            pltpu.make_async_copy(table_hbm.at[src_v.at[j1]],
                                  rows_w, gsem).wait()
            pltpu.sync_copy(rows_w, acc_sh.at[dst_v.at[j1]], add=True)
            pltpu.async_copy(ones_v.at[pl.ds(0, C)],
                             deg_sh.at[dst_v.at[j1]], dsem, add=True)
            return carry
        lax.fori_loop(0, NSUB_PH // 2, _pair, 0)

        pltpu.make_async_copy(table_hbm.at[src_v.at[0]], rows_v,
                              gsem).wait()

        def _drain(t, carry):
            pltpu.make_async_copy(ones_v.at[pl.ds(0, C)],
                                  deg_sh.at[pl.ds(0, C)], dsem).wait()
            return carry
        lax.fori_loop(0, NSUB_PH, _drain, 0)

    plsc.subcore_barrier()

    def _wcopy(t, carry):
        r = sid * RPT + t * WB
        g = cid * NCP + r
        pltpu.sync_copy(acc_sh.at[pl.ds(r, WB)], rows_v.at[pl.ds(0, WB)])
        pltpu.sync_copy(rows_v.at[pl.ds(0, WB)], out_hbm.at[pl.ds(g, WB)])
        return carry
    lax.fori_loop(0, RPT // WB, _wcopy, 0)
    pltpu.sync_copy(deg_sh.at[pl.ds(sid * RPT, RPT)], dstg_v)
    pltpu.sync_copy(dstg_v, deg_hbm.at[pl.ds(cid * NCP + sid * RPT, RPT)])


_seg_sum = functools.partial(
    pl.kernel,
    out_type=(jax.ShapeDtypeStruct((NC * NCP, D), jnp.float32),
              jax.ShapeDtypeStruct((NC * NCP,), jnp.float32)),
    mesh=plsc.VectorSubcoreMesh(core_axis_name="c", subcore_axis_name="s"),
    scratch_types=[
        pltpu.VMEM((NSUB_PH, C), jnp.int32),
        pltpu.VMEM((NSUB_PH, C), jnp.int32),
        pltpu.VMEM((C, D), jnp.float32),
        pltpu.VMEM((C, D), jnp.float32),
        pltpu.VMEM((RPT,), jnp.float32),
        pltpu.VMEM((RPT,), jnp.float32),
        pltpu.VMEM_SHARED((NCP, D), jnp.float32),
        pltpu.VMEM_SHARED((NCP,), jnp.float32),
        pltpu.SemaphoreType.DMA,
        pltpu.SemaphoreType.DMA,
    ],
)(_seg_sum_body)


def _pad_edges(edge):
    src = edge[0].astype(jnp.int32)
    dst = edge[1].astype(jnp.int32)
    pad = E_PAD - E
    src = jnp.concatenate([src, jnp.zeros((pad,), jnp.int32)])
    dst = jnp.concatenate([dst, jnp.full((pad,), N, jnp.int32)])
    return src.reshape(NS * NSUB, C), dst.reshape(NS * NSUB, C)


def _mean_linear_lrelu(acc_ref, deg_ref, w_ref, b_ref, o_ref):
    a = acc_ref[0]
    d = deg_ref[0]
    mean = a / jnp.maximum(d, 1.0)
    z = jnp.dot(mean, w_ref[...], preferred_element_type=jnp.float32)
    z = z + b_ref[...]
    z = jnp.where(z >= 0.0, z, 0.01 * z)
    z = jnp.where(d > 0.0, z, 0.0)
    o_ref[...] = z[:NCR]


def _mean_linear_softmax(acc_ref, deg_ref, w1_ref, b1_ref, wo_ref, bo_ref,
                         o_ref):
    a = acc_ref[0]
    d = deg_ref[0]
    mean = a / jnp.maximum(d, 1.0)
    h = jnp.dot(mean, w1_ref[...], preferred_element_type=jnp.float32)
    h = jnp.where(d > 0.0, h + b1_ref[...], 0.0)
    logits = jnp.dot(h, wo_ref[...], preferred_element_type=jnp.float32)
    logits = logits + bo_ref[...]
    m = jnp.max(logits, axis=-1, keepdims=True)
    e = jnp.exp(logits - m)
    o_ref[...] = (e / jnp.sum(e, axis=-1, keepdims=True))[:NCR]


@jax.jit
def kernel(feat_target, feat_user, edge_interacts, edge_purchases,
           W0_i, b0_i, W0_p, b0_p, W1_i, b1_i, W1_p, b1_p, W_out, b_out):
    src_p, dst_p = _pad_edges(edge_purchases)
    src_i, dst_i = _pad_edges(edge_interacts)

    acc_p, deg_p = _seg_sum(feat_target, src_p, dst_p)
    h_u = pl.pallas_call(
        _mean_linear_lrelu,
        grid=(NC,),
        in_specs=[
            pl.BlockSpec((1, NCP, D), lambda c: (c, 0, 0)),
            pl.BlockSpec((1, NCP, 1), lambda c: (c, 0, 0)),
            pl.BlockSpec((D, D), lambda c: (0, 0)),
            pl.BlockSpec((1, D), lambda c: (0, 0)),
        ],
        out_specs=pl.BlockSpec((NCR, D), lambda c: (c, 0)),
        out_shape=jax.ShapeDtypeStruct((NC * NCR, D), jnp.float32),
    )(acc_p.reshape(NC, NCP, D), deg_p.reshape(NC, NCP, 1),
      W0_p, b0_p.reshape(1, D))

    acc_i, deg_i = _seg_sum(h_u, src_i, dst_i)
    out = pl.pallas_call(
        _mean_linear_softmax,
        grid=(NC,),
        in_specs=[
            pl.BlockSpec((1, NCP, D), lambda c: (c, 0, 0)),
            pl.BlockSpec((1, NCP, 1), lambda c: (c, 0, 0)),
            pl.BlockSpec((D, D), lambda c: (0, 0)),
            pl.BlockSpec((1, D), lambda c: (0, 0)),
            pl.BlockSpec((D, 2), lambda c: (0, 0)),
            pl.BlockSpec((1, 2), lambda c: (0, 0)),
        ],
        out_specs=pl.BlockSpec((NCR, 2), lambda c: (c, 0)),
        out_shape=jax.ShapeDtypeStruct((NC * NCR, 2), jnp.float32),
    )(acc_i.reshape(NC, NCP, D), deg_i.reshape(NC, NCP, 1),
      W1_i, b1_i.reshape(1, D), W_out, b_out.reshape(1, 2))

    return out[:N]


# speedup vs baseline: 1.0025x; 1.0025x over previous
import functools

import jax
import jax.numpy as jnp
from jax import lax
from jax.experimental import pallas as pl
from jax.experimental.pallas import tpu as pltpu
from jax.experimental.pallas import tpu_sc as plsc

N = 10000
E = 160000
D = 128
L = 16
NC = 2
NS = 16
C = 128
E_PAD = 163840
PH = 2
NSUB = E_PAD // (NS * C)
NSUB_PH = NSUB // PH
NCR = 5000
DUMMY = NCR
NCP = 5120
RPT = NCP // NS
WB = 80


def _seg_sum_body(table_hbm, src_hbm, dst_hbm, out_hbm, deg_hbm,
                  src_v, dst_v, rows_v, rows_w, ones_v, dstg_v,
                  acc_sh, deg_sh, gsem, dsem):
    cid = lax.axis_index("c")
    sid = lax.axis_index("s")

    def _zrow(i, carry):
        for k in range(D // L):
            rows_v[i, pl.ds(k * L, L)] = jnp.zeros((L,), jnp.float32)
        return carry
    lax.fori_loop(0, C, _zrow, 0)

    def _zsmall(i, carry):
        dstg_v[pl.ds(i * L, L)] = jnp.zeros((L,), jnp.float32)
        ones_v[pl.ds(i * L, L)] = jnp.ones((L,), jnp.float32)
        return carry
    lax.fori_loop(0, RPT // L, _zsmall, 0)

    def _zcopy(t, carry):
        r = sid * RPT + t * WB
        pltpu.sync_copy(rows_v.at[pl.ds(0, WB)], acc_sh.at[pl.ds(r, WB)])
        return carry
    lax.fori_loop(0, RPT // WB, _zcopy, 0)
    pltpu.sync_copy(dstg_v, deg_sh.at[pl.ds(sid * RPT, RPT)])

    plsc.subcore_barrier()

    base = (cid * NCR).astype(jnp.int32)
    for phase in range(PH):
        row0 = sid * NSUB + phase * NSUB_PH
        pltpu.sync_copy(src_hbm.at[pl.ds(row0, NSUB_PH)], src_v)
        pltpu.sync_copy(dst_hbm.at[pl.ds(row0, NSUB_PH)], dst_v)

        lane = lax.iota(jnp.int32, L)

        def _adj(j, carry):
            for k in range(C // L):
                d = dst_v[j, pl.ds(k * L, L)] - base
                ok = (d >= 0) & (d < NCR)
                dummy = DUMMY + ((lane + (sid + k) * L) % (NCP - NCR))
                dst_v[j, pl.ds(k * L, L)] = jnp.where(ok, d, dummy)
            return carry
        lax.fori_loop(0, NSUB_PH, _adj, 0)

        pltpu.async_copy(table_hbm.at[src_v.at[0]], rows_v, gsem)

        def _pair(p, carry):
            j0 = p * 2
            j1 = j0 + 1
            pltpu.async_copy(table_hbm.at[src_v.at[j1]], rows_w, gsem)
            pltpu.make_async_copy(table_hbm.at[src_v.at[j0]],
                                  rows_v, gsem).wait()
            pltpu.sync_copy(rows_v, acc_sh.at[dst_v.at[j0]], add=True)
            pltpu.async_copy(ones_v.at[pl.ds(0, C)],
                             deg_sh.at[dst_v.at[j0]], dsem, add=True)
            j2 = jnp.minimum(j0 + 2, NSUB_PH - 1)
            pltpu.async_copy(table_hbm.at[src_v.at[j2]], rows_v, gsem)
            pltpu.make_async_copy(table_hbm.at[src_v.at[j1]],
                                  rows_w, gsem).wait()
            pltpu.sync_copy(rows_w, acc_sh.at[dst_v.at[j1]], add=True)
            pltpu.async_copy(ones_v.at[pl.ds(0, C)],
                             deg_sh.at[dst_v.at[j1]], dsem, add=True)
            return carry
        lax.fori_loop(0, NSUB_PH // 2, _pair, 0)

        pltpu.make_async_copy(table_hbm.at[src_v.at[0]], rows_v,
                              gsem).wait()

        def _drain(t, carry):
            pltpu.make_async_copy(ones_v.at[pl.ds(0, C)],
                                  deg_sh.at[pl.ds(0, C)], dsem).wait()
            return carry
        lax.fori_loop(0, NSUB_PH, _drain, 0)

    plsc.subcore_barrier()

    def _wcopy(t, carry):
        r = sid * RPT + t * WB
        g = cid * NCP + r
        pltpu.sync_copy(acc_sh.at[pl.ds(r, WB)], rows_v.at[pl.ds(0, WB)])
        pltpu.sync_copy(rows_v.at[pl.ds(0, WB)], out_hbm.at[pl.ds(g, WB)])
        return carry
    lax.fori_loop(0, RPT // WB, _wcopy, 0)
    pltpu.sync_copy(deg_sh.at[pl.ds(sid * RPT, RPT)], dstg_v)
    pltpu.sync_copy(dstg_v, deg_hbm.at[pl.ds(cid * NCP + sid * RPT, RPT)])


_seg_sum = functools.partial(
    pl.kernel,
    out_type=(jax.ShapeDtypeStruct((NC * NCP, D), jnp.float32),
              jax.ShapeDtypeStruct((NC * NCP,), jnp.float32)),
    mesh=plsc.VectorSubcoreMesh(core_axis_name="c", subcore_axis_name="s"),
    scratch_types=[
        pltpu.VMEM((NSUB_PH, C), jnp.int32),
        pltpu.VMEM((NSUB_PH, C), jnp.int32),
        pltpu.VMEM((C, D), jnp.float32),
        pltpu.VMEM((C, D), jnp.float32),
        pltpu.VMEM((RPT,), jnp.float32),
        pltpu.VMEM((RPT,), jnp.float32),
        pltpu.VMEM_SHARED((NCP, D), jnp.float32),
        pltpu.VMEM_SHARED((NCP,), jnp.float32),
        pltpu.SemaphoreType.DMA,
        pltpu.SemaphoreType.DMA,
    ],
)(_seg_sum_body)


def _pad_edges(edge):
    src = edge[0].astype(jnp.int32)
    dst = edge[1].astype(jnp.int32)
    pad = E_PAD - E
    src = jnp.concatenate([src, jnp.zeros((pad,), jnp.int32)])
    dst = jnp.concatenate([dst, jnp.full((pad,), N, jnp.int32)])
    return src.reshape(NS * NSUB, C), dst.reshape(NS * NSUB, C)


def _mean_linear_lrelu(acc_ref, deg_ref, w_ref, b_ref, o_ref):
    a = acc_ref[0]
    d = deg_ref[0]
    mean = a / jnp.maximum(d, 1.0)
    z = jnp.dot(mean, w_ref[...], preferred_element_type=jnp.float32)
    z = z + b_ref[...]
    z = jnp.where(z >= 0.0, z, 0.01 * z)
    z = jnp.where(d > 0.0, z, 0.0)
    o_ref[...] = z[:NCR]


def _mean_linear_softmax(acc_ref, deg_ref, w1_ref, b1_ref, wo_ref, bo_ref,
                         o_ref):
    a = acc_ref[0]
    d = deg_ref[0]
    mean = a / jnp.maximum(d, 1.0)
    h = jnp.dot(mean, w1_ref[...], preferred_element_type=jnp.float32)
    h = jnp.where(d > 0.0, h + b1_ref[...], 0.0)
    logits = jnp.dot(h, wo_ref[...], preferred_element_type=jnp.float32)
    logits = logits + bo_ref[...]
    m = jnp.max(logits, axis=-1, keepdims=True)
    e = jnp.exp(logits - m)
    o_ref[...] = (e / jnp.sum(e, axis=-1, keepdims=True))[:NCR]


@jax.jit
def kernel(feat_target, feat_user, edge_interacts, edge_purchases,
           W0_i, b0_i, W0_p, b0_p, W1_i, b1_i, W1_p, b1_p, W_out, b_out):
    src_p, dst_p = _pad_edges(edge_purchases)
    src_i, dst_i = _pad_edges(edge_interacts)

    acc_p, deg_p = _seg_sum(feat_target, src_p, dst_p)
    h_u = pl.pallas_call(
        _mean_linear_lrelu,
        grid=(NC,),
        in_specs=[
            pl.BlockSpec((1, NCP, D), lambda c: (c, 0, 0)),
            pl.BlockSpec((1, NCP, 1), lambda c: (c, 0, 0)),
            pl.BlockSpec((D, D), lambda c: (0, 0)),
            pl.BlockSpec((1, D), lambda c: (0, 0)),
        ],
        out_specs=pl.BlockSpec((NCR, D), lambda c: (c, 0)),
        out_shape=jax.ShapeDtypeStruct((NC * NCR, D), jnp.float32),
    )(acc_p.reshape(NC, NCP, D), deg_p.reshape(NC, NCP, 1),
      W0_p, b0_p.reshape(1, D))

    acc_i, deg_i = _seg_sum(h_u, src_i, dst_i)
    out = pl.pallas_call(
        _mean_linear_softmax,
        grid=(NC,),
        in_specs=[
            pl.BlockSpec((1, NCP, D), lambda c: (c, 0, 0)),
            pl.BlockSpec((1, NCP, 1), lambda c: (c, 0, 0)),
            pl.BlockSpec((D, D), lambda c: (0, 0)),
            pl.BlockSpec((1, D), lambda c: (0, 0)),
            pl.BlockSpec((D, 2), lambda c: (0, 0)),
            pl.BlockSpec((1, 2), lambda c: (0, 0)),
        ],
        out_specs=pl.BlockSpec((NCR, 2), lambda c: (c, 0)),
        out_shape=jax.ShapeDtypeStruct((NC * NCR, 2), jnp.float32),
    )(acc_i.reshape(NC, NCP, D), deg_i.reshape(NC, NCP, 1),
      W1_i, b1_i.reshape(1, D), W_out, b_out.reshape(1, 2))

    return out[:N]
